# Initial kernel scaffold; baseline (speedup 1.0000x reference)
import jax
import jax.numpy as jnp
from jax.experimental import pallas as pl


def kernel(x, embed_W, W_ih0, W_hh0, b_ih0, b_hh0, W_ih1, W_hh1, b_ih1, b_hh1, fc_W, fc_b):
    raise NotImplementedError("write your pallas kernel here")


# speedup vs baseline: 3.2302x; 3.2302x over previous
import jax
import jax.numpy as jnp
from jax import lax
from jax.experimental import pallas as pl
from jax.experimental.pallas import tpu as pltpu

_V = 100
_E = 8
_H = 100
_HP = 128
_G = 4 * _HP
_T = 80
_BBLK = 512


def _pad_gates_t(w, in_pad):
    in_dim = w.shape[1]
    w4 = w.reshape(4, _H, in_dim)
    w4 = jnp.pad(w4, ((0, 0), (0, _HP - _H), (0, 0)))
    wt = w4.reshape(4 * _HP, in_dim).T
    if in_pad:
        wt = jnp.pad(wt, ((0, in_pad), (0, 0)))
    return wt


def _pad_bias(b):
    b4 = b.reshape(4, _H)
    b4 = jnp.pad(b4, ((0, 0), (0, _HP - _H)))
    return b4.reshape(1, 4 * _HP)


def _lstm_body(xt_ref, emb_ref, wih0_ref, whh0_ref, b0_ref, wih1_ref,
               whh1_ref, b1_ref, fcw_ref, fcb_ref, out_ref):
    f32 = jnp.float32
    m0 = jnp.dot(emb_ref[...], wih0_ref[...], preferred_element_type=f32)
    b0 = b0_ref[...]
    b1 = b1_ref[...]
    wh0 = whh0_ref[...]
    wi1 = wih1_ref[...]
    wh1 = whh1_ref[...]
    bblk = out_ref.shape[0]

    def gates(g):
        i = jax.nn.sigmoid(g[:, 0:_HP])
        f = jax.nn.sigmoid(g[:, _HP:2 * _HP])
        gg = jnp.tanh(g[:, 2 * _HP:3 * _HP])
        o = jax.nn.sigmoid(g[:, 3 * _HP:4 * _HP])
        return i, f, gg, o

    def step(t, carry):
        h1, c1, h2, c2 = carry
        xt = xt_ref[t]
        oh = (lax.broadcasted_iota(jnp.int32, (_HP, bblk), 0) == xt).astype(f32)
        gx = lax.dot_general(oh, m0, (((0,), (0,)), ((), ())),
                             preferred_element_type=f32)
        g1 = gx + b0 + jnp.dot(h1, wh0, preferred_element_type=f32)
        i1, f1, gg1, o1 = gates(g1)
        c1 = f1 * c1 + i1 * gg1
        h1 = o1 * jnp.tanh(c1)
        g2 = (jnp.dot(h1, wi1, preferred_element_type=f32)
              + jnp.dot(h2, wh1, preferred_element_type=f32) + b1)
        i2, f2, gg2, o2 = gates(g2)
        c2 = f2 * c2 + i2 * gg2
        h2 = o2 * jnp.tanh(c2)
        return h1, c1, h2, c2

    z = jnp.zeros((bblk, _HP), f32)
    h1, c1, h2, c2 = lax.fori_loop(0, _T, step, (z, z, z, z))
    out_ref[...] = jnp.dot(h2, fcw_ref[...], preferred_element_type=f32) + fcb_ref[...]


def _grid_kwargs(batch):
    return dict(
        grid=(batch // _BBLK,),
        in_specs=[
            pl.BlockSpec((_T, 1, _BBLK), lambda j: (0, 0, j)),
            pl.BlockSpec((_HP, _E), lambda j: (0, 0)),
            pl.BlockSpec((_E, _G), lambda j: (0, 0)),
            pl.BlockSpec((_HP, _G), lambda j: (0, 0)),
            pl.BlockSpec((1, _G), lambda j: (0, 0)),
            pl.BlockSpec((_HP, _G), lambda j: (0, 0)),
            pl.BlockSpec((_HP, _G), lambda j: (0, 0)),
            pl.BlockSpec((1, _G), lambda j: (0, 0)),
            pl.BlockSpec((_HP, _V), lambda j: (0, 0)),
            pl.BlockSpec((1, _V), lambda j: (0, 0)),
        ],
        out_specs=pl.BlockSpec((_BBLK, _V), lambda j: (j, 0)),
    )


def kernel(x, embed_W, W_ih0, W_hh0, b_ih0, b_hh0, W_ih1, W_hh1, b_ih1,
           b_hh1, fc_W, fc_b):
    batch = x.shape[0]
    xt = x.T.reshape(_T, 1, batch)
    embp = jnp.pad(embed_W, ((0, _HP - _V), (0, 0)))
    wih0 = _pad_gates_t(W_ih0, 0)
    whh0 = _pad_gates_t(W_hh0, _HP - _H)
    b0 = _pad_bias(b_ih0 + b_hh0)
    wih1 = _pad_gates_t(W_ih1, _HP - _H)
    whh1 = _pad_gates_t(W_hh1, _HP - _H)
    b1 = _pad_bias(b_ih1 + b_hh1)
    fcw = jnp.pad(fc_W.T, ((0, _HP - _H), (0, 0)))
    fcb = fc_b.reshape(1, _V)

    return pl.pallas_call(
        _lstm_body,
        out_shape=jax.ShapeDtypeStruct((batch, _V), jnp.float32),
        compiler_params=pltpu.CompilerParams(
            dimension_semantics=("arbitrary",),
            vmem_limit_bytes=100 * 1024 * 1024,
        ),
        **_grid_kwargs(batch),
    )(xt, embp, wih0, whh0, b0, wih1, whh1, b1, fcw, fcb)


# speedup vs baseline: 4.6040x; 1.4253x over previous
import jax
import jax.numpy as jnp
from jax import lax
from jax.experimental import pallas as pl
from jax.experimental.pallas import tpu as pltpu

_V = 100
_E = 8
_H = 100
_HP = 128
_G = 4 * _HP
_T = 80
_BBLK = 512
_UNROLL = 2

_GATE_SCALE = (0.5, 0.5, 1.0, 0.5)


def _gate_rows(w):
    in_dim = w.shape[1]
    w4 = w.reshape(4, _H, in_dim) * jnp.asarray(_GATE_SCALE, w.dtype)[:, None, None]
    w4 = jnp.pad(w4, ((0, 0), (0, _HP - _H), (0, 0)))
    return w4.reshape(_G, in_dim)


def _gate_bias(b):
    b4 = b.reshape(4, _H) * jnp.asarray(_GATE_SCALE, b.dtype)[:, None]
    b4 = jnp.pad(b4, ((0, 0), (0, _HP - _H)))
    return b4.reshape(_G, 1)


def _lstm_body(xt_ref, embt_ref, wih0_ref, b0_ref, wh0_ref, w2_ref, b1_ref,
               fcw_ref, fcb_ref, out_ref, w1_ref):
    f32 = jnp.float32
    bf16 = jnp.bfloat16
    bblk = out_ref.shape[0]
    nrep = bblk // _HP

    m0 = jnp.dot(wih0_ref[...], embt_ref[...], preferred_element_type=f32)
    w1_ref[:, 0:_HP] = (m0 + b0_ref[...]).astype(bf16)
    w1_ref[:, _HP:2 * _HP] = wh0_ref[...]

    iota = lax.broadcasted_iota(jnp.int32, (_HP, bblk), 0)

    def layer(w_ref, xin, bias, c):
        def gate(k):
            g = jnp.dot(w_ref[k * _HP:(k + 1) * _HP, :], xin,
                        preferred_element_type=f32)
            if bias is not None:
                g = g + pltpu.repeat(bias[k * _HP:(k + 1) * _HP, :], nrep, axis=1)
            return g
        ti = jnp.tanh(gate(0))
        tf = jnp.tanh(gate(1))
        gg = jnp.tanh(gate(2))
        to = jnp.tanh(gate(3))
        i = 0.5 * ti + 0.5
        f = 0.5 * tf + 0.5
        o = 0.5 * to + 0.5
        c = f * c + i * gg
        h = o * jnp.tanh(c)
        return h.astype(bf16), c

    def substep(t, h1, c1, h2, c2):
        xt = xt_ref[t]
        oh = jnp.where(iota == xt, 1.0, 0.0).astype(bf16)
        h1, c1 = layer(w1_ref, jnp.concatenate([oh, h1], axis=0), None, c1)
        h2, c2 = layer(w2_ref, jnp.concatenate([h1, h2], axis=0),
                       b1_ref[...], c2)
        return h1, c1, h2, c2

    def step(tt, carry):
        h1, c1, h2, c2 = carry
        for k in range(_UNROLL):
            h1, c1, h2, c2 = substep(tt * _UNROLL + k, h1, c1, h2, c2)
        return h1, c1, h2, c2

    zb = jnp.zeros((_HP, bblk), bf16)
    zf = jnp.zeros((_HP, bblk), f32)
    h1, c1, h2, c2 = lax.fori_loop(0, _T // _UNROLL, step, (zb, zf, zb, zf))
    out_ref[...] = (
        lax.dot_general(h2, fcw_ref[...], (((0,), (0,)), ((), ())),
                        preferred_element_type=f32)
        + fcb_ref[...])


def kernel(x, embed_W, W_ih0, W_hh0, b_ih0, b_hh0, W_ih1, W_hh1, b_ih1,
           b_hh1, fc_W, fc_b):
    f32 = jnp.float32
    bf16 = jnp.bfloat16
    batch = x.shape[0]
    xt = x.T.reshape(_T, 1, batch)
    embt = jnp.pad(embed_W.T, ((0, 0), (0, _HP - _V)))
    wih0 = _gate_rows(W_ih0)
    b0 = jnp.broadcast_to(_gate_bias(b_ih0 + b_hh0), (_G, _HP))
    wh0 = jnp.pad(_gate_rows(W_hh0),
                  ((0, 0), (0, _HP - _H))).astype(bf16)
    w2 = jnp.concatenate(
        [jnp.pad(_gate_rows(W_ih1), ((0, 0), (0, _HP - _H))),
         jnp.pad(_gate_rows(W_hh1), ((0, 0), (0, _HP - _H)))],
        axis=1).astype(bf16)
    b1 = jnp.broadcast_to(_gate_bias(b_ih1 + b_hh1), (_G, _HP))
    fcw = jnp.pad(fc_W.T, ((0, _HP - _H), (0, 0))).astype(bf16)
    fcb = fc_b.reshape(1, _V)

    return pl.pallas_call(
        _lstm_body,
        out_shape=jax.ShapeDtypeStruct((batch, _V), f32),
        grid=(batch // _BBLK,),
        in_specs=[
            pl.BlockSpec((_T, 1, _BBLK), lambda j: (0, 0, j)),
            pl.BlockSpec((_E, _HP), lambda j: (0, 0)),
            pl.BlockSpec((_G, _E), lambda j: (0, 0)),
            pl.BlockSpec((_G, _HP), lambda j: (0, 0)),
            pl.BlockSpec((_G, _HP), lambda j: (0, 0)),
            pl.BlockSpec((_G, 2 * _HP), lambda j: (0, 0)),
            pl.BlockSpec((_G, _HP), lambda j: (0, 0)),
            pl.BlockSpec((_HP, _V), lambda j: (0, 0)),
            pl.BlockSpec((1, _V), lambda j: (0, 0)),
        ],
        out_specs=pl.BlockSpec((_BBLK, _V), lambda j: (j, 0)),
        scratch_shapes=[pltpu.VMEM((_G, 2 * _HP), bf16)],
        compiler_params=pltpu.CompilerParams(
            dimension_semantics=("arbitrary",),
            vmem_limit_bytes=100 * 1024 * 1024,
        ),
    )(xt, embt, wih0, b0, wh0, w2, b1, fcw, fcb)


# speedup vs baseline: 4.9716x; 1.0799x over previous
import jax
import jax.numpy as jnp
from jax import lax
from jax.experimental import pallas as pl
from jax.experimental.pallas import tpu as pltpu

_V = 100
_E = 8
_H = 100
_HP = 128
_G = 4 * _HP
_NC = _HP // 8
_T = 80
_BBLK = 512
_UNROLL = 2

_GATE_SCALE = (0.5, 0.5, 1.0, 0.5)


def _gate_rows(w):
    in_dim = w.shape[1]
    w4 = w.reshape(4, _H, in_dim) * jnp.asarray(_GATE_SCALE, w.dtype)[:, None, None]
    w4 = jnp.pad(w4, ((0, 0), (0, _HP - _H), (0, 0)))
    w4 = w4.reshape(4, _NC, 8, in_dim).transpose(1, 0, 2, 3)
    return w4.reshape(_G, in_dim)


def _gate_bias(b):
    b4 = b.reshape(4, _H) * jnp.asarray(_GATE_SCALE, b.dtype)[:, None]
    b4 = jnp.pad(b4, ((0, 0), (0, _HP - _H)))
    b4 = b4.reshape(4, _NC, 8).transpose(1, 0, 2)
    return b4.reshape(_G, 1)


def _lstm_body(xt_ref, embt_ref, wih0_ref, b0_ref, wh0_ref, w2_ref, b1_ref,
               fcw_ref, fcb_ref, out_ref, w1_ref):
    f32 = jnp.float32
    bf16 = jnp.bfloat16
    bblk = out_ref.shape[0]
    nrep = bblk // _HP

    m0 = jnp.dot(wih0_ref[...], embt_ref[...], preferred_element_type=f32)
    w1_ref[:, 0:_HP] = (m0 + b0_ref[...]).astype(bf16)
    w1_ref[:, _HP:2 * _HP] = wh0_ref[...]

    iota = lax.broadcasted_iota(jnp.int32, (_HP, bblk), 0)

    def layer(w_ref, xin, bias, c3):
        g = jnp.dot(w_ref[...], xin, preferred_element_type=f32)
        if bias is not None:
            g = g + pltpu.repeat(bias, nrep, axis=1)
        t3 = jnp.tanh(g).reshape(_NC, 32, bblk)
        ti = t3[:, 0:8, :]
        tf = t3[:, 8:16, :]
        tg = t3[:, 16:24, :]
        to = t3[:, 24:32, :]
        c3 = (0.5 * tf + 0.5) * c3 + (0.5 * ti + 0.5) * tg
        h3 = (0.5 * to + 0.5) * jnp.tanh(c3)
        return h3.reshape(_HP, bblk).astype(bf16), c3

    def substep(t, h1, c1, h2, c2):
        xt = xt_ref[t]
        oh = jnp.where(iota == xt, 1.0, 0.0).astype(bf16)
        h1, c1 = layer(w1_ref, jnp.concatenate([oh, h1], axis=0), None, c1)
        h2, c2 = layer(w2_ref, jnp.concatenate([h1, h2], axis=0),
                       b1_ref[...], c2)
        return h1, c1, h2, c2

    def step(tt, carry):
        h1, c1, h2, c2 = carry
        for k in range(_UNROLL):
            h1, c1, h2, c2 = substep(tt * _UNROLL + k, h1, c1, h2, c2)
        return h1, c1, h2, c2

    zb = jnp.zeros((_HP, bblk), bf16)
    zf = jnp.zeros((_NC, 8, bblk), f32)
    h1, c1, h2, c2 = lax.fori_loop(0, _T // _UNROLL, step, (zb, zf, zb, zf))
    out_ref[...] = (
        lax.dot_general(h2, fcw_ref[...], (((0,), (0,)), ((), ())),
                        preferred_element_type=f32)
        + fcb_ref[...])


def kernel(x, embed_W, W_ih0, W_hh0, b_ih0, b_hh0, W_ih1, W_hh1, b_ih1,
           b_hh1, fc_W, fc_b):
    f32 = jnp.float32
    bf16 = jnp.bfloat16
    batch = x.shape[0]
    xt = x.T.reshape(_T, 1, batch)
    embt = jnp.pad(embed_W.T, ((0, 0), (0, _HP - _V)))
    wih0 = _gate_rows(W_ih0)
    b0 = jnp.broadcast_to(_gate_bias(b_ih0 + b_hh0), (_G, _HP))
    wh0 = jnp.pad(_gate_rows(W_hh0),
                  ((0, 0), (0, _HP - _H))).astype(bf16)
    w2 = jnp.concatenate(
        [jnp.pad(_gate_rows(W_ih1), ((0, 0), (0, _HP - _H))),
         jnp.pad(_gate_rows(W_hh1), ((0, 0), (0, _HP - _H)))],
        axis=1).astype(bf16)
    b1 = jnp.broadcast_to(_gate_bias(b_ih1 + b_hh1), (_G, _HP))
    fcw = jnp.pad(fc_W.T, ((0, _HP - _H), (0, 0))).astype(bf16)
    fcb = fc_b.reshape(1, _V)

    return pl.pallas_call(
        _lstm_body,
        out_shape=jax.ShapeDtypeStruct((batch, _V), f32),
        grid=(batch // _BBLK,),
        in_specs=[
            pl.BlockSpec((_T, 1, _BBLK), lambda j: (0, 0, j)),
            pl.BlockSpec((_E, _HP), lambda j: (0, 0)),
            pl.BlockSpec((_G, _E), lambda j: (0, 0)),
            pl.BlockSpec((_G, _HP), lambda j: (0, 0)),
            pl.BlockSpec((_G, _HP), lambda j: (0, 0)),
            pl.BlockSpec((_G, 2 * _HP), lambda j: (0, 0)),
            pl.BlockSpec((_G, _HP), lambda j: (0, 0)),
            pl.BlockSpec((_HP, _V), lambda j: (0, 0)),
            pl.BlockSpec((1, _V), lambda j: (0, 0)),
        ],
        out_specs=pl.BlockSpec((_BBLK, _V), lambda j: (j, 0)),
        scratch_shapes=[pltpu.VMEM((_G, 2 * _HP), bf16)],
        compiler_params=pltpu.CompilerParams(
            dimension_semantics=("arbitrary",),
            vmem_limit_bytes=100 * 1024 * 1024,
        ),
    )(xt, embt, wih0, b0, wh0, w2, b1, fcw, fcb)


# speedup vs baseline: 5.3955x; 1.0853x over previous
import jax
import jax.numpy as jnp
from jax import lax
from jax.experimental import pallas as pl
from jax.experimental.pallas import tpu as pltpu

_V = 100
_VP = 128
_E = 8
_H = 100
_HH = 112
_G = 4 * _HH
_NC = _HH // 8
_K = 240
_T = 80
_BBLK = 512
_UNROLL = 2

_GATE_SCALE = (0.5, 0.5, 1.0, 0.5)


def _gate_rows(w):
    in_dim = w.shape[1]
    w4 = w.reshape(4, _H, in_dim) * jnp.asarray(_GATE_SCALE, w.dtype)[:, None, None]
    w4 = jnp.pad(w4, ((0, 0), (0, _HH - _H), (0, 0)))
    w4 = w4.reshape(4, _NC, 8, in_dim).transpose(1, 0, 2, 3)
    return w4.reshape(_G, in_dim)


def _gate_bias(b):
    b4 = b.reshape(4, _H) * jnp.asarray(_GATE_SCALE, b.dtype)[:, None]
    b4 = jnp.pad(b4, ((0, 0), (0, _HH - _H)))
    b4 = b4.reshape(4, _NC, 8).transpose(1, 0, 2)
    return b4.reshape(_G, 1)


def _hpad(w):
    return jnp.pad(0.5 * w, ((0, 0), (0, _HH - _H)))


def _lstm_body(xt_ref, embt_ref, wih0_ref, b0_ref, wh0_ref, w2_ref,
               fcw_ref, fcb_ref, out_ref, w1_ref):
    f32 = jnp.float32
    bf16 = jnp.bfloat16
    bblk = out_ref.shape[0]

    m0 = jnp.dot(wih0_ref[...], embt_ref[...], preferred_element_type=f32)
    w1_ref[:, 0:_VP] = (m0 + b0_ref[...]).astype(bf16)
    w1_ref[:, _VP:_K] = wh0_ref[...]

    iota = lax.broadcasted_iota(jnp.int32, (_VP, bblk), 0)
    ones16 = jnp.ones((16, bblk), bf16)

    def layer(w_ref, xin, c3):
        g = jnp.dot(w_ref[...], xin, preferred_element_type=f32)
        t3 = jnp.tanh(g).reshape(_NC, 32, bblk)
        ti = t3[:, 0:8, :]
        tf = t3[:, 8:16, :]
        tg = t3[:, 16:24, :]
        to = t3[:, 24:32, :]
        c3 = 0.5 * ((1.0 + tf) * c3 + (1.0 + ti) * tg)
        h3 = (1.0 + to) * jnp.tanh(c3)
        return h3.reshape(_HH, bblk).astype(bf16), c3

    def substep(t, h1, c1, h2, c2):
        xt = xt_ref[t]
        oh = jnp.where(iota == xt, 1.0, 0.0).astype(bf16)
        h1, c1 = layer(w1_ref, jnp.concatenate([oh, h1], axis=0), c1)
        h2, c2 = layer(w2_ref, jnp.concatenate([h1, h2, ones16], axis=0), c2)
        return h1, c1, h2, c2

    def step(tt, carry):
        h1, c1, h2, c2 = carry
        for k in range(_UNROLL):
            h1, c1, h2, c2 = substep(tt * _UNROLL + k, h1, c1, h2, c2)
        return h1, c1, h2, c2

    zb = jnp.zeros((_HH, bblk), bf16)
    zf = jnp.zeros((_NC, 8, bblk), f32)
    h1, c1, h2, c2 = lax.fori_loop(0, _T // _UNROLL, step, (zb, zf, zb, zf))
    out_ref[...] = (
        lax.dot_general(h2, fcw_ref[...], (((0,), (0,)), ((), ())),
                        preferred_element_type=f32)
        + fcb_ref[...])


def kernel(x, embed_W, W_ih0, W_hh0, b_ih0, b_hh0, W_ih1, W_hh1, b_ih1,
           b_hh1, fc_W, fc_b):
    f32 = jnp.float32
    bf16 = jnp.bfloat16
    batch = x.shape[0]
    xt = x.T.reshape(_T, 1, batch)
    embt = jnp.pad(embed_W.T, ((0, 0), (0, _VP - _V)))
    wih0 = _gate_rows(W_ih0)
    b0 = jnp.broadcast_to(_gate_bias(b_ih0 + b_hh0), (_G, _VP))
    wh0 = _hpad(_gate_rows(W_hh0)).astype(bf16)
    bias_block = jnp.pad(_gate_bias(b_ih1 + b_hh1), ((0, 0), (0, 15)))
    w2 = jnp.concatenate(
        [_hpad(_gate_rows(W_ih1)), _hpad(_gate_rows(W_hh1)), bias_block],
        axis=1).astype(bf16)
    fcw = jnp.pad(0.5 * fc_W.T, ((0, _HH - _H), (0, 0))).astype(bf16)
    fcb = fc_b.reshape(1, _V)

    return pl.pallas_call(
        _lstm_body,
        out_shape=jax.ShapeDtypeStruct((batch, _V), f32),
        grid=(batch // _BBLK,),
        in_specs=[
            pl.BlockSpec((_T, 1, _BBLK), lambda j: (0, 0, j)),
            pl.BlockSpec((_E, _VP), lambda j: (0, 0)),
            pl.BlockSpec((_G, _E), lambda j: (0, 0)),
            pl.BlockSpec((_G, _VP), lambda j: (0, 0)),
            pl.BlockSpec((_G, _HH), lambda j: (0, 0)),
            pl.BlockSpec((_G, _K), lambda j: (0, 0)),
            pl.BlockSpec((_HH, _V), lambda j: (0, 0)),
            pl.BlockSpec((1, _V), lambda j: (0, 0)),
        ],
        out_specs=pl.BlockSpec((_BBLK, _V), lambda j: (j, 0)),
        scratch_shapes=[pltpu.VMEM((_G, _K), bf16)],
        compiler_params=pltpu.CompilerParams(
            dimension_semantics=("arbitrary",),
            vmem_limit_bytes=100 * 1024 * 1024,
        ),
    )(xt, embt, wih0, b0, wh0, w2, fcw, fcb)


# speedup vs baseline: 5.9322x; 1.0995x over previous
import jax
import jax.numpy as jnp
from jax import lax
from jax.experimental import pallas as pl
from jax.experimental.pallas import tpu as pltpu

_V = 100
_VP = 128
_E = 8
_H = 100
_HH = 112
_G = 4 * _HH
_NC = _HH // 8
_K = 240
_T = 80
_BBLK = 1024
_BH = 512
_UNROLL = 2

_GATE_SCALE = (0.5, 0.5, 1.0, 0.5)


def _gate_rows(w):
    in_dim = w.shape[1]
    w4 = w.reshape(4, _H, in_dim) * jnp.asarray(_GATE_SCALE, w.dtype)[:, None, None]
    w4 = jnp.pad(w4, ((0, 0), (0, _HH - _H), (0, 0)))
    w4 = w4.reshape(4, _NC, 8, in_dim).transpose(1, 0, 2, 3)
    return w4.reshape(_G, in_dim)


def _gate_bias(b):
    b4 = b.reshape(4, _H) * jnp.asarray(_GATE_SCALE, b.dtype)[:, None]
    b4 = jnp.pad(b4, ((0, 0), (0, _HH - _H)))
    b4 = b4.reshape(4, _NC, 8).transpose(1, 0, 2)
    return b4.reshape(_G, 1)


def _hpad(w):
    return jnp.pad(0.5 * w, ((0, 0), (0, _HH - _H)))


def _lstm_body(xt_ref, embt_ref, wih0_ref, b0_ref, wh0_ref, w2_ref,
               fcw_ref, fcb_ref, out_ref, w1_ref):
    f32 = jnp.float32
    bf16 = jnp.bfloat16
    bblk = out_ref.shape[0]

    m0 = jnp.dot(wih0_ref[...], embt_ref[...], preferred_element_type=f32)
    w1_ref[:, 0:_VP] = (m0 + b0_ref[...]).astype(bf16)
    w1_ref[:, _VP:_K] = wh0_ref[...]

    iota = lax.broadcasted_iota(jnp.int32, (_VP, _BH), 0)
    ones16 = jnp.ones((16, _BH), bf16)

    def layer(w_ref, xin, c3):
        g = jnp.dot(w_ref[...], xin, preferred_element_type=f32)
        t3 = jnp.tanh(g).reshape(_NC, 32, _BH)
        ti = t3[:, 0:8, :]
        tf = t3[:, 8:16, :]
        tg = t3[:, 16:24, :]
        to = t3[:, 24:32, :]
        c3 = 0.5 * ((1.0 + tf) * c3 + (1.0 + ti) * tg)
        h3 = (1.0 + to) * jnp.tanh(c3)
        return h3.reshape(_HH, _BH).astype(bf16), c3

    def half(xt_h, st):
        h1, c1, h2, c2 = st
        oh = jnp.where(iota == xt_h, 1.0, 0.0).astype(bf16)
        h1, c1 = layer(w1_ref, jnp.concatenate([oh, h1], axis=0), c1)
        h2, c2 = layer(w2_ref, jnp.concatenate([h1, h2, ones16], axis=0), c2)
        return h1, c1, h2, c2

    def step(tt, carry):
        sa, sb = carry
        for k in range(_UNROLL):
            xt = xt_ref[tt * _UNROLL + k]
            sa = half(xt[:, 0:_BH], sa)
            sb = half(xt[:, _BH:_BBLK], sb)
        return sa, sb

    zb = jnp.zeros((_HH, _BH), bf16)
    zf = jnp.zeros((_NC, 8, _BH), f32)
    z4 = (zb, zf, zb, zf)
    sa, sb = lax.fori_loop(0, _T // _UNROLL, step, (z4, z4))
    fcw = fcw_ref[...]
    fcb = fcb_ref[...]
    out_ref[0:_BH, :] = (
        lax.dot_general(sa[2], fcw, (((0,), (0,)), ((), ())),
                        preferred_element_type=f32) + fcb)
    out_ref[_BH:_BBLK, :] = (
        lax.dot_general(sb[2], fcw, (((0,), (0,)), ((), ())),
                        preferred_element_type=f32) + fcb)


def kernel(x, embed_W, W_ih0, W_hh0, b_ih0, b_hh0, W_ih1, W_hh1, b_ih1,
           b_hh1, fc_W, fc_b):
    f32 = jnp.float32
    bf16 = jnp.bfloat16
    batch = x.shape[0]
    xt = x.T.reshape(_T, 1, batch)
    embt = jnp.pad(embed_W.T, ((0, 0), (0, _VP - _V)))
    wih0 = _gate_rows(W_ih0)
    b0 = jnp.broadcast_to(_gate_bias(b_ih0 + b_hh0), (_G, _VP))
    wh0 = _hpad(_gate_rows(W_hh0)).astype(bf16)
    bias_block = jnp.pad(_gate_bias(b_ih1 + b_hh1), ((0, 0), (0, 15)))
    w2 = jnp.concatenate(
        [_hpad(_gate_rows(W_ih1)), _hpad(_gate_rows(W_hh1)), bias_block],
        axis=1).astype(bf16)
    fcw = jnp.pad(0.5 * fc_W.T, ((0, _HH - _H), (0, 0))).astype(bf16)
    fcb = fc_b.reshape(1, _V)

    return pl.pallas_call(
        _lstm_body,
        out_shape=jax.ShapeDtypeStruct((batch, _V), f32),
        grid=(batch // _BBLK,),
        in_specs=[
            pl.BlockSpec((_T, 1, _BBLK), lambda j: (0, 0, j)),
            pl.BlockSpec((_E, _VP), lambda j: (0, 0)),
            pl.BlockSpec((_G, _E), lambda j: (0, 0)),
            pl.BlockSpec((_G, _VP), lambda j: (0, 0)),
            pl.BlockSpec((_G, _HH), lambda j: (0, 0)),
            pl.BlockSpec((_G, _K), lambda j: (0, 0)),
            pl.BlockSpec((_HH, _V), lambda j: (0, 0)),
            pl.BlockSpec((1, _V), lambda j: (0, 0)),
        ],
        out_specs=pl.BlockSpec((_BBLK, _V), lambda j: (j, 0)),
        scratch_shapes=[pltpu.VMEM((_G, _K), bf16)],
        compiler_params=pltpu.CompilerParams(
            dimension_semantics=("arbitrary",),
            vmem_limit_bytes=100 * 1024 * 1024,
        ),
    )(xt, embt, wih0, b0, wh0, w2, fcw, fcb)
